# Initial kernel scaffold; baseline (speedup 1.0000x reference)
import functools

import jax
import jax.numpy as jnp
from jax import lax
from jax.experimental import pallas as pl
from jax.experimental.pallas import tpu as pltpu
from jax.experimental.pallas import tpu_sc as plsc

_N = 10000
_E = 320000
_G = 64
_F = 32

_NC = 2
_NS = 16
_NW = _NC * _NS
_EW = _E // _NW
_CH = 80
_NCHUNK = _EW // _CH
_RPT = _N // _NS

_R = 1000
_GRID = _N // _R

_mesh = plsc.VectorSubcoreMesh(core_axis_name="c", subcore_axis_name="s")


@functools.partial(
    pl.kernel,
    out_type=jax.ShapeDtypeStruct((_NW, _N), jnp.float32),
    mesh=_mesh,
    scratch_types=[
        pltpu.VMEM((_EW,), jnp.int32),
        pltpu.VMEM((_N,), jnp.float32),
    ],
)
def _sc_degree(ei_hbm, hist_hbm, dst_v, hist_v):
    c = lax.axis_index("c")
    s = lax.axis_index("s")
    wid = s * _NC + c

    def zero_body(i, carry):
        hist_v[pl.ds(i * 16, 16)] = jnp.zeros((16,), jnp.float32)
        return carry

    lax.fori_loop(0, _N // 16, zero_body, 0)

    pltpu.sync_copy(ei_hbm.at[1, pl.ds(wid * _EW, _EW)], dst_v)

    ones = jnp.ones((16,), jnp.float32)

    def body(e, carry):
        idx = dst_v[pl.ds(e * 16, 16)]
        plsc.addupdate_scatter(hist_v, [idx], ones)
        return carry

    lax.fori_loop(0, _EW // 16, body, 0)

    pltpu.sync_copy(hist_v, hist_hbm.at[wid])


@functools.partial(
    pl.kernel,
    out_type=jax.ShapeDtypeStruct((_NC, _N, _F), jnp.float32),
    mesh=_mesh,
    scratch_types=[
        pltpu.VMEM((_CH,), jnp.int32),
        pltpu.VMEM((_CH,), jnp.int32),
        pltpu.VMEM((_CH, _F), jnp.float32),
        pltpu.VMEM_SHARED((_N, _F), jnp.float32),
        pltpu.SemaphoreType.DMA,
    ],
)
def _sc_scatter(y_hbm, ei_hbm, zeros_hbm, out_hbm, sidx, didx, rows, acc, sem):
    c = lax.axis_index("c")
    s = lax.axis_index("s")
    wid = s * _NC + c

    r0 = s * _RPT
    pltpu.sync_copy(zeros_hbm.at[pl.ds(r0, _RPT)], acc.at[pl.ds(r0, _RPT)])
    plsc.subcore_barrier()

    def body(j, carry):
        base = pl.multiple_of(wid * _EW + j * _CH, _CH)
        pltpu.sync_copy(ei_hbm.at[0, pl.ds(base, _CH)], sidx)
        pltpu.sync_copy(ei_hbm.at[1, pl.ds(base, _CH)], didx)
        pltpu.async_copy(y_hbm.at[sidx], rows, sem).wait()
        pltpu.sync_copy(rows, acc.at[didx], add=True)
        return carry

    lax.fori_loop(0, _NCHUNK, body, 0)

    plsc.subcore_barrier()
    pltpu.sync_copy(acc.at[pl.ds(r0, _RPT)], out_hbm.at[c, pl.ds(r0, _RPT)])


def _mm_body(x_ref, w_ref, hist_ref, y_ref):
    deg = jnp.sum(hist_ref[...], axis=0) + 1.0
    dinv = lax.rsqrt(deg)
    xw = jnp.dot(x_ref[...], w_ref[...], preferred_element_type=jnp.float32)
    y_ref[...] = xw * dinv[:, None]


def _tc_scale_matmul(x, w1, hist):
    return pl.pallas_call(
        _mm_body,
        grid=(_GRID,),
        in_specs=[
            pl.BlockSpec((_R, 128), lambda i: (i, 0)),
            pl.BlockSpec((128, _F), lambda i: (0, 0)),
            pl.BlockSpec((_NW, _R), lambda i: (0, i)),
        ],
        out_specs=pl.BlockSpec((_R, _F), lambda i: (i, 0)),
        out_shape=jax.ShapeDtypeStruct((_N, _F), jnp.float32),
    )(x, w1, hist)


def _combine_body(s_ref, y_ref, hist_ref, b_ref, out_ref):
    deg = jnp.sum(hist_ref[...], axis=0) + 1.0
    dinv = lax.rsqrt(deg)[:, None]
    h = dinv * (s_ref[0] + s_ref[1] + y_ref[...]) + b_ref[...]
    out_ref[...] = h * dinv


def _tc_combine_rescale(s, y, hist, b):
    return pl.pallas_call(
        _combine_body,
        grid=(_GRID,),
        in_specs=[
            pl.BlockSpec((_NC, _R, _F), lambda i: (0, i, 0)),
            pl.BlockSpec((_R, _F), lambda i: (i, 0)),
            pl.BlockSpec((_NW, _R), lambda i: (0, i)),
            pl.BlockSpec((1, _F), lambda i: (0, 0)),
        ],
        out_specs=pl.BlockSpec((_R, _F), lambda i: (i, 0)),
        out_shape=jax.ShapeDtypeStruct((_N, _F), jnp.float32),
    )(s, y, hist, b)


def _head_body(
    s_ref, y_ref, hist_ref, batch_ref, w2_ref, b2_ref, fw1_ref, fb1_ref,
    gamma_ref, beta_ref, fw2_ref, fb2_ref, out_ref, gsum_ref
):
    i = pl.program_id(0)

    @pl.when(i == 0)
    def _():
        gsum_ref[...] = jnp.zeros_like(gsum_ref)

    deg = jnp.sum(hist_ref[...], axis=0) + 1.0
    dinv = lax.rsqrt(deg)[:, None]
    q = dinv * (s_ref[0] + s_ref[1] + y_ref[...])
    b = batch_ref[0, 0, :]
    mask = (b[None, :] == lax.broadcasted_iota(jnp.int32, (_G, _R), 0))
    mask = mask.astype(jnp.float32)
    qa = jnp.concatenate([q, jnp.ones((_R, 8), jnp.float32)], axis=1)
    gsum_ref[...] += jnp.dot(mask, qa, preferred_element_type=jnp.float32)

    @pl.when(i == pl.num_programs(0) - 1)
    def _():
        gs = gsum_ref[...]
        cnt = gs[:, _F:_F + 1]
        gp = gs[:, :_F] / jnp.maximum(cnt, 1.0)
        g = jnp.dot(gp, w2_ref[...], preferred_element_type=jnp.float32)
        g = g + b2_ref[...]
        g = jnp.where(cnt > 0.0, g, 0.0)
        z = jnp.dot(g, fw1_ref[...], preferred_element_type=jnp.float32)
        z = jnp.maximum(z + fb1_ref[...], 0.0)
        mean = jnp.mean(z, axis=0)
        var = jnp.mean((z - mean) ** 2, axis=0)
        z = gamma_ref[...] * (z - mean) * lax.rsqrt(var + 1e-5) + beta_ref[...]
        z = jnp.dot(z, fw2_ref[...], preferred_element_type=jnp.float32)
        out_ref[...] = jnp.maximum(z + fb2_ref[...], 0.0)


def _tc_pool_head(s, y, hist, batch3, w2, b2, fw1, fb1, gamma, beta, fw2, fb2):
    def full(shape):
        return pl.BlockSpec(shape, lambda i: tuple(0 for _ in shape))

    return pl.pallas_call(
        _head_body,
        grid=(_GRID,),
        in_specs=[
            pl.BlockSpec((_NC, _R, _F), lambda i: (0, i, 0)),
            pl.BlockSpec((_R, _F), lambda i: (i, 0)),
            pl.BlockSpec((_NW, _R), lambda i: (0, i)),
            pl.BlockSpec((1, 1, _R), lambda i: (i, 0, 0)),
            full((_F, 64)),
            full((1, 64)),
            full((64, _F)),
            full((1, _F)),
            full((1, _F)),
            full((1, _F)),
            full((_F, 10)),
            full((1, 10)),
        ],
        out_specs=pl.BlockSpec((_G, 10), lambda i: (0, 0)),
        out_shape=jax.ShapeDtypeStruct((_G, 10), jnp.float32),
        scratch_shapes=[pltpu.VMEM((_G, _F + 8), jnp.float32)],
    )(s, y, hist, batch3, w2, b2, fw1, fb1, gamma, beta, fw2, fb2)


def kernel(x, edge_index, batch, W1, b1, W2, b2, fW1, fb1, gamma, beta, fW2, fb2):
    zeros = jnp.zeros((_N, _F), jnp.float32)
    batch3 = batch.reshape(_GRID, 1, _R)

    hist = _sc_degree(edge_index)
    y1 = _tc_scale_matmul(x, W1, hist)
    s1 = _sc_scatter(y1, edge_index, zeros)
    y2 = _tc_combine_rescale(s1, y1, hist, b1.reshape(1, _F))
    s2 = _sc_scatter(y2, edge_index, zeros)
    return _tc_pool_head(
        s2, y2, hist, batch3,
        W2, b2.reshape(1, 64), fW1, fb1.reshape(1, _F),
        gamma.reshape(1, _F), beta.reshape(1, _F), fW2, fb2.reshape(1, 10),
    )


# speedup vs baseline: 19.7508x; 19.7508x over previous
import functools

import jax
import jax.numpy as jnp
from jax import lax
from jax.experimental import pallas as pl
from jax.experimental.pallas import tpu as pltpu
from jax.experimental.pallas import tpu_sc as plsc

_N = 10000
_E = 320000
_G = 64
_F = 32
_FD = 8

_NC = 2
_NS = 16
_NW = _NC * _NS
_EW = _E // _NW
_CH = 80
_NCHUNK = _EW // _CH
_RPT = 632

_R = 1000
_GRID = _N // _R


@functools.cache
def _sc_kernels():
    mesh = plsc.VectorSubcoreMesh(
        core_axis_name="c", subcore_axis_name="s", num_cores=_NC, num_subcores=_NS
    )

    @functools.partial(
        pl.kernel,
        out_type=jax.ShapeDtypeStruct((_NC, _N, _FD), jnp.float32),
        mesh=mesh,
        scratch_types=[
            pltpu.VMEM((_CH,), jnp.int32),
            pltpu.VMEM((_CH, _FD), jnp.float32),
            pltpu.VMEM_SHARED((_N, _FD), jnp.float32),
        ],
    )
    def sc_degree(dst_hbm, zeros_hbm, out_hbm, didx, ones_v, acc):
        c = lax.axis_index("c")
        s = lax.axis_index("s")
        wid = s * _NC + c

        def ones_body(i, carry):
            ones_v[pl.ds(i * 2, 2), :] = jnp.ones((16,), jnp.float32).reshape(2, 8)
            return carry

        lax.fori_loop(0, _CH // 2, ones_body, 0)

        r0 = pl.multiple_of(jnp.minimum(s * _RPT, _N - _RPT), 8)
        pltpu.sync_copy(zeros_hbm.at[pl.ds(r0, _RPT)], acc.at[pl.ds(r0, _RPT)])
        plsc.subcore_barrier()

        def body(j, carry):
            base = pl.multiple_of(wid * _EW + j * _CH, _CH)
            pltpu.sync_copy(dst_hbm.at[pl.ds(base, _CH)], didx)
            pltpu.sync_copy(ones_v, acc.at[didx], add=True)
            return carry

        lax.fori_loop(0, _NCHUNK, body, 0)

        plsc.subcore_barrier()
        pltpu.sync_copy(acc.at[pl.ds(r0, _RPT)], out_hbm.at[c, pl.ds(r0, _RPT)])

    @functools.partial(
        pl.kernel,
        out_type=jax.ShapeDtypeStruct((_NC, _N, _F), jnp.float32),
        mesh=mesh,
        scratch_types=[
            pltpu.VMEM((_CH,), jnp.int32),
            pltpu.VMEM((_CH,), jnp.int32),
            pltpu.VMEM((_CH, _F), jnp.float32),
            pltpu.VMEM_SHARED((_N, _F), jnp.float32),
            pltpu.VMEM_SHARED((_N, _F), jnp.float32),
            pltpu.SemaphoreType.DMA,
        ],
    )
    def sc_scatter(y_hbm, src_hbm, dst_hbm, zeros_hbm, out_hbm, sidx, didx, rows,
                   acc, y_sh, sem):
        c = lax.axis_index("c")
        s = lax.axis_index("s")
        wid = s * _NC + c

        r0 = pl.multiple_of(jnp.minimum(s * _RPT, _N - _RPT), 8)
        pltpu.sync_copy(zeros_hbm.at[pl.ds(r0, _RPT)], acc.at[pl.ds(r0, _RPT)])
        pltpu.sync_copy(y_hbm.at[pl.ds(r0, _RPT)], y_sh.at[pl.ds(r0, _RPT)])
        plsc.subcore_barrier()

        def body(j, carry):
            base = pl.multiple_of(wid * _EW + j * _CH, _CH)
            pltpu.sync_copy(src_hbm.at[pl.ds(base, _CH)], sidx)
            pltpu.sync_copy(dst_hbm.at[pl.ds(base, _CH)], didx)
            pltpu.async_copy(y_sh.at[sidx], rows, sem).wait()
            pltpu.sync_copy(rows, acc.at[didx], add=True)
            return carry

        lax.fori_loop(0, _NCHUNK, body, 0)

        plsc.subcore_barrier()
        pltpu.sync_copy(acc.at[pl.ds(r0, _RPT)], out_hbm.at[c, pl.ds(r0, _RPT)])

    return sc_degree, sc_scatter


def _dinv(hist_blk):
    deg = hist_blk[0, :, 0:1] + hist_blk[1, :, 0:1] + 1.0
    return lax.rsqrt(deg)


def _mm_body(x_ref, w_ref, hist_ref, y_ref):
    xw = jnp.dot(x_ref[...], w_ref[...], preferred_element_type=jnp.float32)
    y_ref[...] = xw * _dinv(hist_ref[...])


def _tc_scale_matmul(x, w1, hist):
    return pl.pallas_call(
        _mm_body,
        grid=(_GRID,),
        in_specs=[
            pl.BlockSpec((_R, 128), lambda i: (i, 0)),
            pl.BlockSpec((128, _F), lambda i: (0, 0)),
            pl.BlockSpec((_NC, _R, _FD), lambda i: (0, i, 0)),
        ],
        out_specs=pl.BlockSpec((_R, _F), lambda i: (i, 0)),
        out_shape=jax.ShapeDtypeStruct((_N, _F), jnp.float32),
    )(x, w1, hist)


def _combine_body(s_ref, y_ref, hist_ref, b_ref, out_ref):
    dinv = _dinv(hist_ref[...])
    h = dinv * (s_ref[0] + s_ref[1] + y_ref[...]) + b_ref[...]
    out_ref[...] = h * dinv


def _tc_combine_rescale(s, y, hist, b):
    return pl.pallas_call(
        _combine_body,
        grid=(_GRID,),
        in_specs=[
            pl.BlockSpec((_NC, _R, _F), lambda i: (0, i, 0)),
            pl.BlockSpec((_R, _F), lambda i: (i, 0)),
            pl.BlockSpec((_NC, _R, _FD), lambda i: (0, i, 0)),
            pl.BlockSpec((1, _F), lambda i: (0, 0)),
        ],
        out_specs=pl.BlockSpec((_R, _F), lambda i: (i, 0)),
        out_shape=jax.ShapeDtypeStruct((_N, _F), jnp.float32),
    )(s, y, hist, b)


def _head_body(
    s_ref, y_ref, hist_ref, batch_ref, w2_ref, b2_ref, fw1_ref, fb1_ref,
    gamma_ref, beta_ref, fw2_ref, fb2_ref, out_ref, gsum_ref
):
    i = pl.program_id(0)

    @pl.when(i == 0)
    def _():
        gsum_ref[...] = jnp.zeros_like(gsum_ref)

    dinv = _dinv(hist_ref[...])
    q = dinv * (s_ref[0] + s_ref[1] + y_ref[...])
    b = batch_ref[0, 0, :]
    mask = (b[None, :] == lax.broadcasted_iota(jnp.int32, (_G, _R), 0))
    mask = mask.astype(jnp.float32)
    qa = jnp.concatenate([q, jnp.ones((_R, 8), jnp.float32)], axis=1)
    gsum_ref[...] += jnp.dot(mask, qa, preferred_element_type=jnp.float32)

    @pl.when(i == pl.num_programs(0) - 1)
    def _():
        gs = gsum_ref[...]
        cnt = gs[:, _F:_F + 1]
        gp = gs[:, :_F] / jnp.maximum(cnt, 1.0)
        g = jnp.dot(gp, w2_ref[...], preferred_element_type=jnp.float32)
        g = g + b2_ref[...]
        g = jnp.where(cnt > 0.0, g, 0.0)
        z = jnp.dot(g, fw1_ref[...], preferred_element_type=jnp.float32)
        z = jnp.maximum(z + fb1_ref[...], 0.0)
        mean = jnp.mean(z, axis=0)
        var = jnp.mean((z - mean) ** 2, axis=0)
        z = gamma_ref[...] * (z - mean) * lax.rsqrt(var + 1e-5) + beta_ref[...]
        z = jnp.dot(z, fw2_ref[...], preferred_element_type=jnp.float32)
        out_ref[...] = jnp.maximum(z + fb2_ref[...], 0.0)


def _tc_pool_head(s, y, hist, batch3, w2, b2, fw1, fb1, gamma, beta, fw2, fb2):
    def full(shape):
        return pl.BlockSpec(shape, lambda i: tuple(0 for _ in shape))

    return pl.pallas_call(
        _head_body,
        grid=(_GRID,),
        in_specs=[
            pl.BlockSpec((_NC, _R, _F), lambda i: (0, i, 0)),
            pl.BlockSpec((_R, _F), lambda i: (i, 0)),
            pl.BlockSpec((_NC, _R, _FD), lambda i: (0, i, 0)),
            pl.BlockSpec((1, 1, _R), lambda i: (i, 0, 0)),
            full((_F, 64)),
            full((1, 64)),
            full((64, _F)),
            full((1, _F)),
            full((1, _F)),
            full((1, _F)),
            full((_F, 10)),
            full((1, 10)),
        ],
        out_specs=pl.BlockSpec((_G, 10), lambda i: (0, 0)),
        out_shape=jax.ShapeDtypeStruct((_G, 10), jnp.float32),
        scratch_shapes=[pltpu.VMEM((_G, _F + 8), jnp.float32)],
    )(s, y, hist, batch3, w2, b2, fw1, fb1, gamma, beta, fw2, fb2)


def kernel(x, edge_index, batch, W1, b1, W2, b2, fW1, fb1, gamma, beta, fW2, fb2):
    zeros = jnp.zeros((_N, _F), jnp.float32)
    zeros8 = jnp.zeros((_N, _FD), jnp.float32)
    batch3 = batch.reshape(_GRID, 1, _R)

    src = edge_index[0]
    dst = edge_index[1]
    sc_degree, sc_scatter = _sc_kernels()
    hist = sc_degree(dst, zeros8)
    y1 = _tc_scale_matmul(x, W1, hist)
    s1 = sc_scatter(y1, src, dst, zeros)
    y2 = _tc_combine_rescale(s1, y1, hist, b1.reshape(1, _F))
    s2 = sc_scatter(y2, src, dst, zeros)
    return _tc_pool_head(
        s2, y2, hist, batch3,
        W2, b2.reshape(1, 64), fW1, fb1.reshape(1, _F),
        gamma.reshape(1, _F), beta.reshape(1, _F), fW2, fb2.reshape(1, 10),
    )
